# baseline (speedup 1.0000x reference)
import functools

import jax
import jax.numpy as jnp
from jax import lax
from jax.experimental import pallas as pl
from jax.experimental.pallas import tpu as pltpu
from jax.experimental.pallas import tpu_sc as plsc

_TOKENS = 8192
_H = 2048
_I = 5632
_B = 512


@functools.cache
def _make_sc_gather():
    info = plsc.get_sparse_core_info()
    nw = info.num_cores * info.num_subcores
    b_per_w = _B // nw
    mesh = plsc.VectorSubcoreMesh(core_axis_name="c", subcore_axis_name="s")

    @functools.partial(
        pl.kernel,
        mesh=mesh,
        out_type=jax.ShapeDtypeStruct((_B, _H), jnp.float32),
        scratch_types=[
            pltpu.VMEM((b_per_w,), jnp.int32),
            pltpu.VMEM((b_per_w, _H), jnp.float32),
            pltpu.SemaphoreType.DMA,
        ],
    )
    def gather_kernel(x_hbm, idx_hbm, out_hbm, idx_v, rows_v, sem):
        wid = lax.axis_index("s") * info.num_cores + lax.axis_index("c")
        base = wid * b_per_w
        pltpu.sync_copy(idx_hbm.at[pl.ds(base, b_per_w)], idx_v)
        pltpu.async_copy(x_hbm.at[idx_v], rows_v, sem).wait()
        pltpu.sync_copy(rows_v, out_hbm.at[pl.ds(base, b_per_w)])

    return gather_kernel


_BI = 512
_NI = _I // _BI
_HK = _H // 2

_NT = (((1,), (1,)), ((), ()))
_PREC = lax.Precision.DEFAULT


def _dot_nt(a, b):
    return lax.dot_general(a, b, _NT, precision=_PREC,
                           preferred_element_type=jnp.float32)


def _ffn_body(xs_ref, w_ref, wg_hbm, wu_hbm, wd_hbm, out_ref,
              ga_buf, gb_buf, up_buf, dn_buf, sga, sgb, sup, sdn):
    def fetch_gate(j):
        s = j % 3
        for k in range(2):
            cols = pl.ds(k * _HK, _HK)
            pltpu.make_async_copy(
                wg_hbm.at[pl.ds(j * _BI, _BI), cols],
                ga_buf.at[s, :, cols], sga.at[s, k]).start()
            pltpu.make_async_copy(
                wg_hbm.at[pl.ds(_I + j * _BI, _BI), cols],
                gb_buf.at[s, :, cols], sgb.at[s, k]).start()
            pltpu.make_async_copy(
                wu_hbm.at[pl.ds(j * _BI, _BI), cols],
                up_buf.at[s, :, cols], sup.at[s, k]).start()

    def fetch_dn(j):
        s = j % 2
        pltpu.make_async_copy(wd_hbm.at[:, pl.ds(j * _BI, _BI)],
                              dn_buf.at[s], sdn.at[s]).start()

    fetch_gate(0)
    fetch_dn(0)
    fetch_gate(1)
    fetch_dn(1)
    fetch_gate(2)

    for j in range(_NI):
        s = j % 3
        ds_ = j % 2
        parts = []
        for row0, buf, sem in ((j * _BI, ga_buf, sga),
                               (_I + j * _BI, gb_buf, sgb),
                               (j * _BI, up_buf, sup)):
            src = wg_hbm if buf is not up_buf else wu_hbm
            acc = None
            for k in range(2):
                cols = pl.ds(k * _HK, _HK)
                pltpu.make_async_copy(
                    src.at[pl.ds(row0, _BI), cols],
                    buf.at[s, :, cols], sem.at[s, k]).wait()
                d = _dot_nt(xs_ref[:, cols], buf[s, :, cols])
                acc = d if acc is None else acc + d
            parts.append(acc)
        ga, gb, up = parts
        if j + 3 < _NI:
            fetch_gate(j + 3)
        h = ga * jax.nn.sigmoid(gb) * up * w_ref[...]
        pltpu.make_async_copy(wd_hbm.at[:, pl.ds(j * _BI, _BI)],
                              dn_buf.at[ds_], sdn.at[ds_]).wait()
        part = _dot_nt(h, dn_buf[ds_])
        if j + 2 < _NI:
            fetch_dn(j + 2)
        if j == 0:
            out_ref[...] = part
        else:
            out_ref[...] += part


def _ffn(xs, weight, W_gate, W_up, W_down):
    return pl.pallas_call(
        _ffn_body,
        in_specs=[
            pl.BlockSpec(memory_space=pltpu.VMEM),
            pl.BlockSpec(memory_space=pltpu.VMEM),
            pl.BlockSpec(memory_space=pl.ANY),
            pl.BlockSpec(memory_space=pl.ANY),
            pl.BlockSpec(memory_space=pl.ANY),
        ],
        out_specs=pl.BlockSpec(memory_space=pltpu.VMEM),
        out_shape=jax.ShapeDtypeStruct((_B, _H), jnp.float32),
        scratch_shapes=[
            pltpu.VMEM((3, _BI, _H), jnp.float32),
            pltpu.VMEM((3, _BI, _H), jnp.float32),
            pltpu.VMEM((3, _BI, _H), jnp.float32),
            pltpu.VMEM((2, _H, _BI), jnp.float32),
            pltpu.SemaphoreType.DMA((3, 2)),
            pltpu.SemaphoreType.DMA((3, 2)),
            pltpu.SemaphoreType.DMA((3, 2)),
            pltpu.SemaphoreType.DMA((2,)),
        ],
    )(xs, weight, W_gate, W_up, W_down)


def kernel(x, top_x, weight, W_gate, W_up, W_down):
    xs = _make_sc_gather()(x, top_x.astype(jnp.int32))
    return _ffn(xs, weight, W_gate, W_up, W_down)


# speedup vs baseline: 1.0859x; 1.0859x over previous
import functools

import jax
import jax.numpy as jnp
from jax import lax
from jax.experimental import pallas as pl
from jax.experimental.pallas import tpu as pltpu
from jax.experimental.pallas import tpu_sc as plsc

_TOKENS = 8192
_H = 2048
_I = 5632
_B = 512


@functools.cache
def _make_sc_gather():
    info = plsc.get_sparse_core_info()
    nw = info.num_cores * info.num_subcores
    b_per_w = _B // nw
    mesh = plsc.VectorSubcoreMesh(core_axis_name="c", subcore_axis_name="s")

    @functools.partial(
        pl.kernel,
        mesh=mesh,
        out_type=jax.ShapeDtypeStruct((_B, _H), jnp.float32),
        scratch_types=[
            pltpu.VMEM((b_per_w,), jnp.int32),
            pltpu.VMEM((b_per_w, _H), jnp.float32),
            pltpu.SemaphoreType.DMA,
        ],
    )
    def gather_kernel(x_hbm, idx_hbm, out_hbm, idx_v, rows_v, sem):
        wid = lax.axis_index("s") * info.num_cores + lax.axis_index("c")
        base = wid * b_per_w
        pltpu.sync_copy(idx_hbm.at[pl.ds(base, b_per_w)], idx_v)
        pltpu.async_copy(x_hbm.at[idx_v], rows_v, sem).wait()
        pltpu.sync_copy(rows_v, out_hbm.at[pl.ds(base, b_per_w)])

    return gather_kernel


_BI = 512
_NI = _I // _BI

_NT = (((1,), (1,)), ((), ()))


def _ffn_body(xs_ref, w_ref, wga_ref, wgb_ref, wup_ref, wdn_ref, out_ref):
    i = pl.program_id(0)
    xb = xs_ref[...]
    ga = lax.dot_general(xb, wga_ref[...], _NT,
                         precision=lax.Precision.DEFAULT,
                         preferred_element_type=jnp.float32)
    gb = lax.dot_general(xb, wgb_ref[...], _NT,
                         precision=lax.Precision.DEFAULT,
                         preferred_element_type=jnp.float32)
    up = lax.dot_general(xb, wup_ref[...], _NT,
                         precision=lax.Precision.DEFAULT,
                         preferred_element_type=jnp.float32)
    h = ga * jax.nn.sigmoid(gb) * up
    part = lax.dot_general(h, wdn_ref[...], _NT,
                           precision=lax.Precision.DEFAULT,
                           preferred_element_type=jnp.float32)

    @pl.when(i == 0)
    def _():
        out_ref[...] = part

    @pl.when(jnp.logical_and(i > 0, i < _NI - 1))
    def _():
        out_ref[...] += part

    @pl.when(i == _NI - 1)
    def _():
        out_ref[...] = (out_ref[...] + part) * w_ref[...]


def _ffn(xs, weight, W_gate, W_up, W_down):
    return pl.pallas_call(
        _ffn_body,
        grid=(_NI,),
        in_specs=[
            pl.BlockSpec((_B, _H), lambda i: (0, 0)),
            pl.BlockSpec((_B, 1), lambda i: (0, 0)),
            pl.BlockSpec((_BI, _H), lambda i: (i, 0)),
            pl.BlockSpec((_BI, _H), lambda i: (i + _NI, 0)),
            pl.BlockSpec((_BI, _H), lambda i: (i, 0)),
            pl.BlockSpec((_H, _BI), lambda i: (0, i)),
        ],
        out_specs=pl.BlockSpec((_B, _H), lambda i: (0, 0)),
        out_shape=jax.ShapeDtypeStruct((_B, _H), jnp.float32),
        compiler_params=pltpu.CompilerParams(
            dimension_semantics=("arbitrary",),
        ),
    )(xs, weight, W_gate, W_gate, W_up, W_down)


def kernel(x, top_x, weight, W_gate, W_up, W_down):
    xs = _make_sc_gather()(x, top_x.astype(jnp.int32))
    return _ffn(xs, weight, W_gate, W_up, W_down)
